# Initial kernel scaffold; baseline (speedup 1.0000x reference)
import jax
import jax.numpy as jnp
from jax.experimental import pallas as pl


def kernel(Graph, norm_h, norm_L, norm_W, norm_P, norm_N, T, ln_g, ln_b, W0, We0, al0, ar0, ae0, b0, W1, We1, al1, ar1, ae1, b1, Wl, bl):
    raise NotImplementedError("write your pallas kernel here")


# speedup vs baseline: 1.7032x; 1.7032x over previous
import functools

import jax
import jax.numpy as jnp
from jax.experimental import pallas as pl


def _lrelu(x, s):
    return jnp.where(x >= 0, x, s * x)


def _gat_kernel(nfr_ref, g_ref, tp_ref,
                ln_g_ref, ln_b_ref,
                w0_ref, al0_ref, ar0_ref, ae0_ref, we0_ref, b0_ref,
                w1_ref, al1_ref, ar1_ref, ae1_ref, we1_ref, b1_ref,
                h1_ref, *, J, N, H, F0, ED):
    f32 = jnp.float32

    x = nfr_ref[...]
    mu = jnp.mean(x, axis=-1, keepdims=True)
    var = jnp.mean((x - mu) ** 2, axis=-1, keepdims=True)
    xn = (x - mu) / jnp.sqrt(var + 1e-5) * ln_g_ref[...] + ln_b_ref[...]

    g = g_ref[...]
    mask = g > 0
    tp = tp_ref[...]

    def gat_layer(feat, w_ref, al_ref, ar_ref, ae_ref, we_ref, b_ref, D):
        ft = jax.lax.dot_general(
            feat, w_ref[...], (((1,), (0,)), ((), ())),
            preferred_element_type=f32)
        acc = jnp.zeros((N, D), f32)
        for h in range(H):
            sl = slice(h * D, (h + 1) * D)
            fth = ft[:, sl]
            al = al_ref[h:h + 1, :]
            ar = ar_ref[h:h + 1, :]
            ae = ae_ref[h:h + 1, :]
            few = we_ref[:, sl]
            el = jax.lax.dot_general(
                fth[:J, :], al, (((1,), (1,)), ((), ())),
                preferred_element_type=f32)
            er = jax.lax.dot_general(
                ar, fth, (((1,), (1,)), ((), ())),
                preferred_element_type=f32)
            eec = jnp.sum(few * ae, keepdims=True)
            lg = _lrelu(el + er + tp * eec, 0.2)
            lg = jnp.where(mask, lg, -1e9)
            mx = jnp.max(lg, axis=0, keepdims=True)
            ex = jnp.where(mask, jnp.exp(lg - mx), 0.0)
            den = jnp.sum(ex, axis=0, keepdims=True)
            alpha = ex / jnp.where(den > 0, den, 1.0)
            outh = jax.lax.dot_general(
                alpha, fth[:J, :], (((0,), (0,)), ((), ())),
                preferred_element_type=f32)
            eagg = jax.lax.dot_general(
                alpha * tp, jnp.ones((J, 1), f32), (((0,), (0,)), ((), ())),
                preferred_element_type=f32)
            o = outh + eagg * few + b_ref[:, sl]
            acc = acc + _lrelu(o, 0.01)
        return acc * (1.0 / H)

    h0 = gat_layer(xn, w0_ref, al0_ref, ar0_ref, ae0_ref, we0_ref, b0_ref, F0)
    h1 = gat_layer(h0, w1_ref, al1_ref, ar1_ref, ae1_ref, we1_ref, b1_ref, ED)
    h1_ref[...] = h1


def _final_kernel(x_ref, wl_ref, bl_ref, o_ref):
    acc = jax.lax.dot_general(
        x_ref[...], wl_ref[...], (((1,), (0,)), ((), ())),
        preferred_element_type=jnp.float32)
    o_ref[...] = _lrelu(acc + bl_ref[...], 0.01)


def kernel(Graph, norm_h, norm_L, norm_W, norm_P, norm_N, T, ln_g, ln_b,
           W0, We0, al0, ar0, ae0, b0, W1, We1, al1, ar1, ae1, b1, Wl, bl):
    f32 = jnp.float32
    BS, J = norm_h.shape
    N = Graph.shape[1] // J
    H, F0 = al0.shape
    ED = al1.shape[1]

    G3 = Graph.reshape(BS, J, N)
    other = jnp.concatenate([norm_W, norm_P, norm_N], axis=1)
    jobf = jnp.concatenate(
        [norm_h[..., None], norm_L[..., None],
         jnp.broadcast_to(other[:, None, :], (BS, J, 3))], axis=-1)
    nfr = jnp.concatenate(
        [jobf, jnp.zeros((BS, N - J, 5), f32)], axis=1)
    Tp = jnp.concatenate(
        [T, jnp.zeros((BS, J, N - J), f32)], axis=-1)

    ln_g2 = ln_g.reshape(1, 5)
    ln_b2 = ln_b.reshape(1, 5)
    b0r = b0.reshape(1, H * F0)
    b1r = b1.reshape(1, H * ED)
    blr = bl.reshape(1, ED)

    def rep(shape):
        return pl.BlockSpec(shape, lambda i: (0,) * len(shape))

    gat = pl.pallas_call(
        functools.partial(_gat_kernel, J=J, N=N, H=H, F0=F0, ED=ED),
        grid=(BS,),
        in_specs=[
            pl.BlockSpec((None, N, 5), lambda i: (i, 0, 0)),
            pl.BlockSpec((None, J, N), lambda i: (i, 0, 0)),
            pl.BlockSpec((None, J, N), lambda i: (i, 0, 0)),
            rep((1, 5)), rep((1, 5)),
            rep((5, H * F0)), rep((H, F0)), rep((H, F0)), rep((H, F0)),
            rep((1, H * F0)), rep((1, H * F0)),
            rep((F0, H * ED)), rep((H, ED)), rep((H, ED)), rep((H, ED)),
            rep((1, H * ED)), rep((1, H * ED)),
        ],
        out_specs=pl.BlockSpec((None, N, ED), lambda i: (i, 0, 0)),
        out_shape=jax.ShapeDtypeStruct((BS, N, ED), f32),
    )
    h1 = gat(nfr, G3, Tp, ln_g2, ln_b2,
             W0, al0, ar0, ae0, We0, b0r,
             W1, al1, ar1, ae1, We1, b1r)

    BR = 64
    fin = pl.pallas_call(
        _final_kernel,
        grid=(BS // BR,),
        in_specs=[
            pl.BlockSpec((BR, N * ED), lambda i: (i, 0)),
            pl.BlockSpec((N * ED, ED), lambda i: (0, 0)),
            pl.BlockSpec((1, ED), lambda i: (0, 0)),
        ],
        out_specs=pl.BlockSpec((BR, ED), lambda i: (i, 0)),
        out_shape=jax.ShapeDtypeStruct((BS, ED), f32),
    )
    return fin(h1.reshape(BS, N * ED), Wl, blr)


# speedup vs baseline: 1.9973x; 1.1727x over previous
import functools

import jax
import jax.numpy as jnp
from jax.experimental import pallas as pl


def _lrelu(x, s):
    return jnp.maximum(x, s * x)


def _gat_kernel(nfr_ref, g_ref, tp_ref,
                ln_g_ref, ln_b_ref,
                w0_ref, al0_ref, ar0_ref, ae0_ref, we0_ref, b0_ref,
                w1_ref, al1_ref, ar1_ref, ae1_ref, we1_ref, b1_ref,
                h1_ref, *, B, J, N, H, F0, ED):
    f32 = jnp.float32

    def gat_layer(feat, mask, tp, w_ref, al_ref, ar_ref, ae_ref, we_ref,
                  b_ref, D):
        ft = jax.lax.dot_general(
            feat, w_ref[...], (((1,), (0,)), ((), ())),
            preferred_element_type=f32)
        acc = None
        for h in range(H):
            sl = slice(h * D, (h + 1) * D)
            fth = ft[:, sl]
            al = al_ref[h:h + 1, :]
            ar = ar_ref[h:h + 1, :]
            ae = ae_ref[h:h + 1, :]
            few = we_ref[:, sl]
            el = jax.lax.dot_general(
                fth[:J, :], al, (((1,), (1,)), ((), ())),
                preferred_element_type=f32)
            er = jax.lax.dot_general(
                ar, fth, (((1,), (1,)), ((), ())),
                preferred_element_type=f32)
            eec = jnp.sum(few * ae, keepdims=True)
            lg = _lrelu(el + er + tp * eec, 0.2)
            ex = jnp.where(mask, jnp.exp(lg), 0.0)
            den = jnp.sum(ex, axis=0, keepdims=True)
            alpha = ex / jnp.where(den > 0, den, 1.0)
            outh = jax.lax.dot_general(
                alpha, fth[:J, :], (((0,), (0,)), ((), ())),
                preferred_element_type=f32)
            eagg = jax.lax.dot_general(
                alpha * tp, jnp.ones((J, 1), f32), (((0,), (0,)), ((), ())),
                preferred_element_type=f32)
            o = _lrelu(outh + eagg * few + b_ref[:, sl], 0.01)
            acc = o if acc is None else acc + o
        return acc * (1.0 / H)

    for b in range(B):
        x = nfr_ref[b]
        mu = jnp.mean(x, axis=-1, keepdims=True)
        var = jnp.mean((x - mu) ** 2, axis=-1, keepdims=True)
        xn = (x - mu) / jnp.sqrt(var + 1e-5) * ln_g_ref[...] + ln_b_ref[...]

        mask = g_ref[b] > 0
        tp = tp_ref[b]

        h0 = gat_layer(xn, mask, tp, w0_ref, al0_ref, ar0_ref, ae0_ref,
                       we0_ref, b0_ref, F0)
        h1 = gat_layer(h0, mask, tp, w1_ref, al1_ref, ar1_ref, ae1_ref,
                       we1_ref, b1_ref, ED)
        h1_ref[b] = h1


def _final_kernel(x_ref, wl_ref, bl_ref, o_ref):
    acc = jax.lax.dot_general(
        x_ref[...], wl_ref[...], (((1,), (0,)), ((), ())),
        preferred_element_type=jnp.float32)
    o_ref[...] = _lrelu(acc + bl_ref[...], 0.01)


def kernel(Graph, norm_h, norm_L, norm_W, norm_P, norm_N, T, ln_g, ln_b,
           W0, We0, al0, ar0, ae0, b0, W1, We1, al1, ar1, ae1, b1, Wl, bl):
    f32 = jnp.float32
    BS, J = norm_h.shape
    N = Graph.shape[1] // J
    H, F0 = al0.shape
    ED = al1.shape[1]
    B = 4

    G3 = Graph.reshape(BS, J, N)
    other = jnp.concatenate([norm_W, norm_P, norm_N], axis=1)
    jobf = jnp.concatenate(
        [norm_h[..., None], norm_L[..., None],
         jnp.broadcast_to(other[:, None, :], (BS, J, 3))], axis=-1)
    nfr = jnp.concatenate(
        [jobf, jnp.zeros((BS, N - J, 5), f32)], axis=1)
    Tp = jnp.concatenate(
        [T, jnp.zeros((BS, J, N - J), f32)], axis=-1)

    ln_g2 = ln_g.reshape(1, 5)
    ln_b2 = ln_b.reshape(1, 5)
    b0r = b0.reshape(1, H * F0)
    b1r = b1.reshape(1, H * ED)
    blr = bl.reshape(1, ED)

    def rep(shape):
        return pl.BlockSpec(shape, lambda i: (0,) * len(shape))

    gat = pl.pallas_call(
        functools.partial(_gat_kernel, B=B, J=J, N=N, H=H, F0=F0, ED=ED),
        grid=(BS // B,),
        in_specs=[
            pl.BlockSpec((B, N, 5), lambda i: (i, 0, 0)),
            pl.BlockSpec((B, J, N), lambda i: (i, 0, 0)),
            pl.BlockSpec((B, J, N), lambda i: (i, 0, 0)),
            rep((1, 5)), rep((1, 5)),
            rep((5, H * F0)), rep((H, F0)), rep((H, F0)), rep((H, F0)),
            rep((1, H * F0)), rep((1, H * F0)),
            rep((F0, H * ED)), rep((H, ED)), rep((H, ED)), rep((H, ED)),
            rep((1, H * ED)), rep((1, H * ED)),
        ],
        out_specs=pl.BlockSpec((B, N, ED), lambda i: (i, 0, 0)),
        out_shape=jax.ShapeDtypeStruct((BS, N, ED), f32),
    )
    h1 = gat(nfr, G3, Tp, ln_g2, ln_b2,
             W0, al0, ar0, ae0, We0, b0r,
             W1, al1, ar1, ae1, We1, b1r)

    BR = 64
    fin = pl.pallas_call(
        _final_kernel,
        grid=(BS // BR,),
        in_specs=[
            pl.BlockSpec((BR, N * ED), lambda i: (i, 0)),
            pl.BlockSpec((N * ED, ED), lambda i: (0, 0)),
            pl.BlockSpec((1, ED), lambda i: (0, 0)),
        ],
        out_specs=pl.BlockSpec((BR, ED), lambda i: (i, 0)),
        out_shape=jax.ShapeDtypeStruct((BS, ED), f32),
    )
    return fin(h1.reshape(BS, N * ED), Wl, blr)
